# baseline (speedup 1.0000x reference)
import functools

import jax
import jax.numpy as jnp
from jax import lax
from jax.experimental import pallas as pl
from jax.experimental.pallas import tpu as pltpu
from jax.experimental.pallas import tpu_sc as plsc


def _make_sc_kernel(B, D, b_per_w, NC, NW, L):
    CHUNK = 128
    n_chunks = b_per_w // CHUNK
    vecs_per_row = D // L

    mesh = plsc.VectorSubcoreMesh(core_axis_name="c", subcore_axis_name="s")

    @functools.partial(
        pl.kernel,
        mesh=mesh,
        compiler_params=pltpu.CompilerParams(use_tc_tiling_on_sc=False),
        out_type=jax.ShapeDtypeStruct((NW, L), jnp.float32),
        scratch_types=[
            pltpu.VMEM((b_per_w,), jnp.int32),
            pltpu.VMEM((b_per_w, D), jnp.float32),
            pltpu.VMEM((b_per_w, D), jnp.float32),
            pltpu.VMEM((L,), jnp.float32),
            pltpu.SemaphoreType.DMA,
        ],
    )
    def sc_kernel(x_hbm, idx_hbm, tab_hbm, out_hbm, idx_v, xs_v, rows_v, acc_v,
                  sem):
        wid = lax.axis_index("s") * NC + lax.axis_index("c")
        base = wid * b_per_w

        pltpu.sync_copy(idx_hbm.at[pl.ds(base, b_per_w)], idx_v)
        gathers = [
            pltpu.async_copy(
                tab_hbm.at[idx_v.at[pl.ds(j * CHUNK, CHUNK)]],
                rows_v.at[pl.ds(j * CHUNK, CHUNK)],
                sem,
            )
            for j in range(n_chunks)
        ]
        pltpu.sync_copy(x_hbm.at[pl.ds(base, b_per_w)], xs_v)
        for g in gathers:
            g.wait()

        zero = jnp.zeros((L,), jnp.float32)

        def body(i, accs):
            out = []
            for c in range(vecs_per_row):
                sl = pl.ds(c * L, L)
                d = xs_v[i, sl] - rows_v[i, sl]
                out.append(accs[c] + d * d)
            return tuple(out)

        accs = lax.fori_loop(0, b_per_w, body, (zero,) * vecs_per_row)
        total = accs[0]
        for c in range(1, vecs_per_row):
            total = total + accs[c]
        acc_v[...] = total
        pltpu.sync_copy(acc_v, out_hbm.at[wid])

    return sc_kernel


def kernel(x, labels, centers):
    B, D = x.shape
    info = plsc.get_sparse_core_info()
    NC, NS, L = info.num_cores, info.num_subcores, info.num_lanes
    NW = NC * NS
    b_per_w = B // NW

    sc_kernel = _make_sc_kernel(B, D, b_per_w, NC, NW, L)
    partials = sc_kernel(x, labels.astype(jnp.int32), centers)
    return jnp.sum(partials) / (B * D)


# speedup vs baseline: 1.9808x; 1.9808x over previous
import functools

import jax
import jax.numpy as jnp
from jax import lax
from jax.experimental import pallas as pl
from jax.experimental.pallas import tpu as pltpu
from jax.experimental.pallas import tpu_sc as plsc


def _make_sc_kernel(B, D, V, NC, NW, L):
    ROW_MAIN = (V // 128) * 128
    TAIL = V - ROW_MAIN
    F_PER_W = D // NW
    CHUNK = 2048
    n_chunks = B // CHUNK
    inv_n = 1.0 / (B * D)

    mesh = plsc.VectorSubcoreMesh(core_axis_name="c", subcore_axis_name="s")

    @functools.partial(
        pl.kernel,
        mesh=mesh,
        compiler_params=pltpu.CompilerParams(use_tc_tiling_on_sc=True,
                                             needs_layout_passes=False),
        out_type=jax.ShapeDtypeStruct((NW, 8, 128), jnp.float32),
        scratch_types=[
            pltpu.VMEM((ROW_MAIN + 128,), jnp.float32),
            pltpu.VMEM((CHUNK,), jnp.int32),
            pltpu.VMEM((CHUNK,), jnp.int32),
            pltpu.VMEM((CHUNK,), jnp.float32),
            pltpu.VMEM((CHUNK,), jnp.float32),
            pltpu.VMEM((8, 128), jnp.float32),
            pltpu.SemaphoreType.DMA,
            pltpu.SemaphoreType.DMA,
        ],
    )
    def sc_kernel(x_hbm, idx_hbm, tab_hbm, tail_hbm, out_hbm, row_v, lab0,
                  lab1, xv0, xv1, ob_v, lsem, xsem):
        wid = lax.axis_index("s") * NC + lax.axis_index("c")
        labs = (lab0, lab1)
        xvs = (xv0, xv1)
        acc = jnp.zeros((L,), jnp.float32)

        for fi in range(F_PER_W):
            f = F_PER_W * wid + fi
            t = lax.shift_right_logical(f, 3)
            r = lax.bitwise_and(f, 7)

            pltpu.sync_copy(tab_hbm.at[t, r, pl.ds(0, ROW_MAIN)],
                            row_v.at[pl.ds(0, ROW_MAIN)])
            if TAIL:
                pltpu.sync_copy(tail_hbm.at[t, r, pl.ds(0, 128)],
                                row_v.at[pl.ds(ROW_MAIN, 128)])

            def fire(c, buf):
                lc = pltpu.async_copy(
                    idx_hbm.at[pl.ds(c * CHUNK, CHUNK)], labs[buf], lsem)
                xc = pltpu.async_copy(
                    x_hbm.at[t, r, pl.ds(c * CHUNK, CHUNK)], xvs[buf], xsem)
                return lc, xc

            pending = fire(0, 0)
            for c in range(n_chunks):
                buf = c % 2
                for p in pending:
                    p.wait()
                if c + 1 < n_chunks:
                    pending = fire(c + 1, 1 - buf)

                lab_v = labs[buf]
                xv = xvs[buf]

                def body(g, a):
                    sl = pl.ds(g * L, L)
                    cv = plsc.load_gather(row_v, [lab_v[sl]])
                    d = xv[sl] - cv
                    return a + d * d

                acc = lax.fori_loop(0, CHUNK // L, body, acc)

        for rr in range(8):
            for h in range(128 // L):
                ob_v[rr, pl.ds(h * L, L)] = jnp.zeros((L,), jnp.float32)
        ob_v[0, pl.ds(0, L)] = acc * inv_n
        pltpu.sync_copy(ob_v, out_hbm.at[wid])

    return sc_kernel


def kernel(x, labels, centers):
    B, D = x.shape
    V = centers.shape[0]
    info = plsc.get_sparse_core_info()
    NC, NS, L = info.num_cores, info.num_subcores, info.num_lanes
    NW = NC * NS

    sc_kernel = _make_sc_kernel(B, D, V, NC, NW, L)
    row_main = (V // 128) * 128
    tail = jnp.pad(centers[row_main:].T, ((0, 0), (0, 128 - (V - row_main))))
    partials = sc_kernel(
        x.T.reshape(D // 8, 8, B),
        labels.astype(jnp.int32),
        centers.T.reshape(D // 8, 8, V),
        tail.reshape(D // 8, 8, 128),
    )
    return jnp.sum(partials)
